# Initial kernel scaffold; baseline (speedup 1.0000x reference)
import jax
import jax.numpy as jnp
from jax import lax
from jax.experimental import pallas as pl
from jax.experimental.pallas import tpu as pltpu
from jax.experimental.pallas import tpu_sc as plsc

N = 10000
D = 128
E = 320000
NC = 2
NS = 16
NW = NC * NS
CHUNK = 128
NCHUNK = 79
EPAD = NW * NCHUNK * CHUNK
ROWS_PER_TILE = N // NS


def _sc_spmm(x, srcs, dsts, vals):
    mesh = plsc.VectorSubcoreMesh(core_axis_name="c", subcore_axis_name="s")

    def body(x_hbm, src_hbm, dst_hbm, val_hbm, out_hbm,
             acc_shared, src_v, dst_v, val_v, rows_v, sem):
        c = lax.axis_index("c")
        s = lax.axis_index("s")
        wid = c * NS + s

        zeros16 = jnp.zeros((16,), jnp.float32)

        def zero_row(e, carry):
            for j in range(D // 16):
                rows_v[e, pl.ds(j * 16, 16)] = zeros16
            return carry

        lax.fori_loop(0, CHUNK, zero_row, 0)
        for t in range(5):
            pltpu.sync_copy(rows_v.at[pl.ds(0, 125)],
                            acc_shared.at[pl.ds(s * ROWS_PER_TILE + t * 125, 125)])
        plsc.subcore_barrier()

        def chunk_body(ci, carry):
            pltpu.sync_copy(src_hbm.at[wid, ci], src_v)
            pltpu.sync_copy(dst_hbm.at[wid, ci], dst_v)
            pltpu.sync_copy(val_hbm.at[wid, ci], val_v)
            pltpu.async_copy(x_hbm.at[src_v], rows_v, sem).wait()

            def scale(e, inner):
                v = val_v[e]
                vv = jnp.full((16,), v, jnp.float32)
                for j in range(D // 16):
                    sl = pl.ds(j * 16, 16)
                    rows_v[e, sl] = rows_v[e, sl] * vv
                return inner

            lax.fori_loop(0, CHUNK, scale, 0)
            pltpu.sync_copy(rows_v, acc_shared.at[dst_v], add=True)
            return carry

        lax.fori_loop(0, NCHUNK, chunk_body, 0)
        plsc.subcore_barrier()

        pltpu.sync_copy(acc_shared.at[pl.ds(s * ROWS_PER_TILE, ROWS_PER_TILE)],
                        out_hbm.at[c, pl.ds(s * ROWS_PER_TILE, ROWS_PER_TILE)])

    return pl.kernel(
        body,
        out_type=jax.ShapeDtypeStruct((NC, N, D), jnp.float32),
        mesh=mesh,
        scratch_types=[
            pltpu.VMEM_SHARED((N, D), jnp.float32),
            pltpu.VMEM((CHUNK,), jnp.int32),
            pltpu.VMEM((CHUNK,), jnp.int32),
            pltpu.VMEM((CHUNK,), jnp.float32),
            pltpu.VMEM((CHUNK, D), jnp.float32),
            pltpu.SemaphoreType.DMA,
        ],
    )(x, srcs, dsts, vals)


def _tc_mix(p0, p1, i_1, w, beta_arr):
    BLK = 1000

    def body(b_ref, p0_ref, p1_ref, i_ref, w_ref, o_ref):
        b = b_ref[0]
        m = (1.0 - b) * i_ref[...] + b * w_ref[...]
        o_ref[...] = jnp.dot(p0_ref[...] + p1_ref[...], m,
                             preferred_element_type=jnp.float32)

    return pl.pallas_call(
        body,
        grid=(N // BLK,),
        in_specs=[
            pl.BlockSpec(memory_space=pltpu.SMEM),
            pl.BlockSpec((BLK, D), lambda i: (i, 0)),
            pl.BlockSpec((BLK, D), lambda i: (i, 0)),
            pl.BlockSpec((D, D), lambda i: (0, 0)),
            pl.BlockSpec((D, D), lambda i: (0, 0)),
        ],
        out_specs=pl.BlockSpec((BLK, D), lambda i: (i, 0)),
        out_shape=jax.ShapeDtypeStruct((N, D), jnp.float32),
    )(beta_arr, p0, p1, i_1, w)


def kernel(X, edge_index, edge_vals, I_1, W, gamma, beta):
    src = edge_index[0].astype(jnp.int32)
    dst = edge_index[1].astype(jnp.int32)
    vals = edge_vals.astype(jnp.float32)
    pad = EPAD - E
    src = jnp.concatenate([src, jnp.zeros((pad,), jnp.int32)])
    dst = jnp.concatenate([dst, jnp.zeros((pad,), jnp.int32)])
    vals = jnp.concatenate([vals, jnp.zeros((pad,), jnp.float32)])
    srcs = src.reshape(NW, NCHUNK, CHUNK)
    dsts = dst.reshape(NW, NCHUNK, CHUNK)
    valsr = vals.reshape(NW, NCHUNK, CHUNK)

    partial = _sc_spmm(X, srcs, dsts, valsr)
    beta_arr = jnp.asarray(beta, jnp.float32).reshape(1)
    return _tc_mix(partial[0], partial[1], I_1, W, beta_arr)


# speedup vs baseline: 3.4712x; 3.4712x over previous
import jax
import jax.numpy as jnp
from jax import lax
from jax.experimental import pallas as pl
from jax.experimental.pallas import tpu as pltpu
from jax.experimental.pallas import tpu_sc as plsc

N = 10000
D = 128
E = 320000
NC = 2
NS = 16
NW = NC * NS
CHUNK = 128
NCHUNK = 79
EPAD = NW * NCHUNK * CHUNK
NPAD = 10240
ROWS_PER_TILE = NPAD // NS


def _sc_spmm(x, srcs, dsts, vals):
    mesh = plsc.VectorSubcoreMesh(core_axis_name="c", subcore_axis_name="s")

    def body(x_hbm, src_hbm, dst_hbm, val_hbm, out_hbm,
             acc_shared, src_v, dst_v, val_v, rows_v, sem):
        c = lax.axis_index("c")
        s = lax.axis_index("s")
        wid = c * NS + s

        zeros16 = jnp.zeros((16,), jnp.float32)

        def zero_row(e, carry):
            for j in range(D // 16):
                rows_v[e, pl.ds(j * 16, 16)] = zeros16
            return carry

        lax.fori_loop(0, CHUNK, zero_row, 0)
        for t in range(ROWS_PER_TILE // CHUNK):
            pltpu.sync_copy(rows_v,
                            acc_shared.at[pl.ds(s * ROWS_PER_TILE + t * CHUNK, CHUNK)])
        plsc.subcore_barrier()

        def chunk_body(ci, carry):
            pltpu.sync_copy(src_hbm.at[wid, ci], src_v)
            pltpu.sync_copy(dst_hbm.at[wid, ci], dst_v)
            pltpu.sync_copy(val_hbm.at[wid, ci], val_v)
            pltpu.async_copy(x_hbm.at[src_v], rows_v, sem).wait()

            def scale(g, inner):
                base = g * 16
                vv = val_v[pl.ds(base, 16)]
                for k in range(16):
                    vb = jnp.full((16,), vv[k], jnp.float32)
                    for j in range(D // 16):
                        sl = pl.ds(j * 16, 16)
                        rows_v[base + k, sl] = rows_v[base + k, sl] * vb
                return inner

            lax.fori_loop(0, CHUNK // 16, scale, 0)
            pltpu.sync_copy(rows_v, acc_shared.at[dst_v], add=True)
            return carry

        lax.fori_loop(0, NCHUNK, chunk_body, 0)
        plsc.subcore_barrier()

        pltpu.sync_copy(acc_shared.at[pl.ds(s * ROWS_PER_TILE, ROWS_PER_TILE)],
                        out_hbm.at[c, pl.ds(s * ROWS_PER_TILE, ROWS_PER_TILE)])

    return pl.kernel(
        body,
        out_type=jax.ShapeDtypeStruct((NC, NPAD, D), jnp.float32),
        mesh=mesh,
        scratch_types=[
            pltpu.VMEM_SHARED((NPAD, D), jnp.float32),
            pltpu.VMEM((CHUNK,), jnp.int32),
            pltpu.VMEM((CHUNK,), jnp.int32),
            pltpu.VMEM((CHUNK,), jnp.float32),
            pltpu.VMEM((CHUNK, D), jnp.float32),
            pltpu.SemaphoreType.DMA,
        ],
    )(x, srcs, dsts, vals)


def _tc_mix(p0, p1, i_1, w, beta_arr):
    BLK = 640

    def body(b_ref, p0_ref, p1_ref, i_ref, w_ref, o_ref):
        b = b_ref[0]
        m = (1.0 - b) * i_ref[...] + b * w_ref[...]
        o_ref[...] = jnp.dot(p0_ref[...] + p1_ref[...], m,
                             preferred_element_type=jnp.float32)

    return pl.pallas_call(
        body,
        grid=(NPAD // BLK,),
        in_specs=[
            pl.BlockSpec(memory_space=pltpu.SMEM),
            pl.BlockSpec((BLK, D), lambda i: (i, 0)),
            pl.BlockSpec((BLK, D), lambda i: (i, 0)),
            pl.BlockSpec((D, D), lambda i: (0, 0)),
            pl.BlockSpec((D, D), lambda i: (0, 0)),
        ],
        out_specs=pl.BlockSpec((BLK, D), lambda i: (i, 0)),
        out_shape=jax.ShapeDtypeStruct((NPAD, D), jnp.float32),
    )(beta_arr, p0, p1, i_1, w)


def kernel(X, edge_index, edge_vals, I_1, W, gamma, beta):
    src = edge_index[0].astype(jnp.int32)
    dst = edge_index[1].astype(jnp.int32)
    vals = edge_vals.astype(jnp.float32)
    pad = EPAD - E
    src = jnp.concatenate([src, jnp.zeros((pad,), jnp.int32)])
    dst = jnp.concatenate([dst, jnp.zeros((pad,), jnp.int32)])
    vals = jnp.concatenate([vals, jnp.zeros((pad,), jnp.float32)])
    srcs = src.reshape(NW, NCHUNK, CHUNK)
    dsts = dst.reshape(NW, NCHUNK, CHUNK)
    valsr = vals.reshape(NW, NCHUNK, CHUNK)

    partial = _sc_spmm(X, srcs, dsts, valsr)
    beta_arr = jnp.asarray(beta, jnp.float32).reshape(1)
    return _tc_mix(partial[0], partial[1], I_1, W, beta_arr)[:N]
